# baseline (speedup 1.0000x reference)
import functools

import jax
import jax.numpy as jnp
from jax import lax
from jax.experimental import pallas as pl
from jax.experimental.pallas import tpu as pltpu
from jax.experimental.pallas import tpu_sc as plsc

T = 2048
E = 1024
H = 16
DH = 64
PH = 4
NK = 128
TK = 8
DQ = 1024
BT = 256
NTB = T // BT


def _k1_body(x_ref, g_ref, b_ref, w_ref, bias_ref, out_ref):
    x = x_ref[...]
    mu = jnp.mean(x, axis=1, keepdims=True)
    var = jnp.mean((x - mu) ** 2, axis=1, keepdims=True)
    xn = (x - mu) * lax.rsqrt(var + 1e-5) * g_ref[...] + b_ref[...]
    out_ref[...] = (
        jnp.dot(xn.astype(jnp.bfloat16), w_ref[...],
                preferred_element_type=jnp.float32)
        + bias_ref[...]
    ).astype(jnp.bfloat16)


def _k1(x, g, b, W, bias):
    return pl.pallas_call(
        _k1_body,
        grid=(NTB,),
        compiler_params=pltpu.CompilerParams(
            dimension_semantics=("parallel",)
        ),
        in_specs=[
            pl.BlockSpec((BT, E), lambda i: (i, 0)),
            pl.BlockSpec((1, E), lambda i: (0, 0)),
            pl.BlockSpec((1, E), lambda i: (0, 0)),
            pl.BlockSpec((E, 3 * E), lambda i: (0, 0)),
            pl.BlockSpec((1, 3 * E), lambda i: (0, 0)),
        ],
        out_specs=pl.BlockSpec((BT, 3 * E), lambda i: (i, 0)),
        out_shape=jax.ShapeDtypeStruct((T, 3 * E), jnp.bfloat16),
    )(x, g, b, W, bias)


BT2 = 1024
NTB2 = T // BT2


def _k2_body(q_ref, k_ref, v_ref, attn_ref, ctx_ref):
    q2 = q_ref[...]
    k2 = k_ref[...]
    v2 = v_ref[...]
    ctx_parts = []
    for half in range(2):
        c = slice(half * DH, (half + 1) * DH)
        s = lax.dot_general(
            q2[:, c], k2[:, c], (((1,), (1,)), ((), ())),
            preferred_element_type=jnp.float32,
        ) * 0.125
        m = jnp.max(s, axis=1, keepdims=True)
        p = jnp.exp(s - m)
        inv = 1.0 / jnp.sum(p, axis=1, keepdims=True)
        attn_ref[half] = p * inv
        ctx = lax.dot_general(
            p.astype(jnp.bfloat16), v2[:, c], (((1,), (0,)), ((), ())),
            preferred_element_type=jnp.float32,
        )
        ctx_parts.append(ctx * inv)
    ctx_ref[...] = jnp.concatenate(ctx_parts, axis=1).astype(jnp.bfloat16)


def _k2(qkv):
    return pl.pallas_call(
        _k2_body,
        grid=(H // 2, NTB2),
        compiler_params=pltpu.CompilerParams(
            dimension_semantics=("parallel", "parallel")
        ),
        in_specs=[
            pl.BlockSpec((BT2, 2 * DH), lambda h, i: (i, h)),
            pl.BlockSpec((T, 2 * DH), lambda h, i: (0, H // 2 + h)),
            pl.BlockSpec((T, 2 * DH), lambda h, i: (0, H + h)),
        ],
        out_specs=[
            pl.BlockSpec((2, BT2, T), lambda h, i: (h, i, 0)),
            pl.BlockSpec((BT2, 2 * DH), lambda h, i: (i, h)),
        ],
        out_shape=[
            jax.ShapeDtypeStruct((H, T, T), jnp.float32),
            jax.ShapeDtypeStruct((T, E), jnp.bfloat16),
        ],
    )(qkv, qkv, qkv)


def _k3_body(ctx_ref, wo_ref, bo_ref, res_ref, g_ref, b_ref, wq_ref,
             hid_ref, q_ref, st_ref):
    i = pl.program_id(0)
    attn_out = (
        jnp.dot(ctx_ref[...], wo_ref[...], preferred_element_type=jnp.float32)
        + bo_ref[...]
    )
    hid = attn_out + res_ref[...]
    hid_ref[...] = hid
    mu = jnp.mean(hid, axis=1, keepdims=True)
    var = jnp.mean((hid - mu) ** 2, axis=1, keepdims=True)
    xn = (hid - mu) * lax.rsqrt(var + 1e-5) * g_ref[...] + b_ref[...]
    q = jnp.dot(xn.astype(jnp.bfloat16), wq_ref[...],
                preferred_element_type=jnp.float32)
    q_ref[...] = q
    s1 = jnp.sum(q, axis=0, keepdims=True)
    s2 = jnp.sum(q * q, axis=0, keepdims=True)
    blk = jnp.concatenate([s1, s2], axis=0)

    @pl.when(i == 0)
    def _():
        st_ref[...] = blk

    @pl.when(i != 0)
    def _():
        st_ref[...] = st_ref[...] + blk


def _k3(ctx, Wo, bo, resid, g, b, Wq):
    return pl.pallas_call(
        _k3_body,
        grid=(NTB,),
        in_specs=[
            pl.BlockSpec((BT, E), lambda i: (i, 0)),
            pl.BlockSpec((E, E), lambda i: (0, 0)),
            pl.BlockSpec((1, E), lambda i: (0, 0)),
            pl.BlockSpec((BT, E), lambda i: (i, 0)),
            pl.BlockSpec((1, E), lambda i: (0, 0)),
            pl.BlockSpec((1, E), lambda i: (0, 0)),
            pl.BlockSpec((E, DQ), lambda i: (0, 0)),
        ],
        out_specs=[
            pl.BlockSpec((BT, E), lambda i: (i, 0)),
            pl.BlockSpec((BT, DQ), lambda i: (i, 0)),
            pl.BlockSpec((2, DQ), lambda i: (0, 0)),
        ],
        out_shape=[
            jax.ShapeDtypeStruct((T, E), jnp.float32),
            jax.ShapeDtypeStruct((T, DQ), jnp.float32),
            jax.ShapeDtypeStruct((2, DQ), jnp.float32),
        ],
    )(ctx, Wo, bo, resid, g, b, Wq)


def _topk8t(s, iota):
    n = s.shape[0]
    cur = s
    vals, poss = [], []
    for _ in range(TK):
        m = jnp.max(cur, axis=0, keepdims=True)
        p = jnp.min(
            jnp.where(cur == m, iota, jnp.int32(n)), axis=0, keepdims=True
        )
        vals.append(m)
        poss.append(p)
        cur = jnp.where(iota == p, jnp.float32(-3e38), cur)
    return jnp.concatenate(vals, axis=0), jnp.concatenate(poss, axis=0)


def _k4_body(q_ref, st_ref, g_ref, b_ref, keys_ref, w_ref, vi_ref):
    q = q_ref[...]
    st = st_ref[...]
    mu = st[0:1, :] * (1.0 / T)
    ms = st[1:2, :] * (1.0 / T)
    var = ms - mu * mu
    qn = (q - mu) * lax.rsqrt(var + 1e-5) * g_ref[...] + b_ref[...]
    iota_nk = lax.broadcasted_iota(jnp.int32, (NK, BT), 0)
    iota_cc = lax.broadcasted_iota(jnp.int32, (TK * TK, BT), 0)
    w_parts, vi_parts = [], []
    for h in range(PH):
        q1 = qn[:, h * NK:(h + 1) * NK]
        q2 = qn[:, PH * NK + h * NK: PH * NK + (h + 1) * NK]
        d1 = lax.dot_general(
            keys_ref[0, h], q1, (((1,), (1,)), ((), ())),
            preferred_element_type=jnp.float32,
        )
        d2 = lax.dot_general(
            keys_ref[1, h], q2, (((1,), (1,)), ((), ())),
            preferred_element_type=jnp.float32,
        )
        s1v, i1v = _topk8t(d1, iota_nk)
        s2v, i2v = _topk8t(d2, iota_nk)
        combs, combi = [], []
        for a in range(TK):
            combs.append(s1v[a:a + 1, :] + s2v)
            combi.append(i1v[a:a + 1, :] * NK + i2v)
        comb = jnp.concatenate(combs, axis=0)
        ci = jnp.concatenate(combi, axis=0)
        fs, fpos = _topk8t(comb, iota_cc)
        sels = []
        for j in range(TK):
            pj = fpos[j:j + 1, :]
            sels.append(
                jnp.min(
                    jnp.where(iota_cc == pj, ci, jnp.int32(1 << 30)),
                    axis=0,
                    keepdims=True,
                )
            )
        vi_h = jnp.concatenate(sels, axis=0)
        ex = jnp.exp(fs - fs[0:1, :])
        wgt = ex * (1.0 / jnp.sum(ex, axis=0, keepdims=True))
        w_parts.append(wgt)
        vi_parts.append(vi_h)
    w_ref[...] = jnp.concatenate(w_parts, axis=0)
    vi_ref[...] = jnp.concatenate(vi_parts, axis=0)


def _k4(queries, stats, g, b, keysP):
    return pl.pallas_call(
        _k4_body,
        grid=(NTB,),
        compiler_params=pltpu.CompilerParams(
            dimension_semantics=("parallel",)
        ),
        in_specs=[
            pl.BlockSpec((BT, DQ), lambda i: (i, 0)),
            pl.BlockSpec((2, DQ), lambda i: (0, 0)),
            pl.BlockSpec((1, DQ), lambda i: (0, 0)),
            pl.BlockSpec((1, DQ), lambda i: (0, 0)),
            pl.BlockSpec((2, PH, NK, NK), lambda i: (0, 0, 0, 0)),
        ],
        out_specs=[
            pl.BlockSpec((PH * TK, BT), lambda i: (0, i)),
            pl.BlockSpec((PH * TK, BT), lambda i: (0, i)),
        ],
        out_shape=[
            jax.ShapeDtypeStruct((PH * TK, T), jnp.float32),
            jax.ShapeDtypeStruct((PH * TK, T), jnp.int32),
        ],
    )(queries, stats, g, b, keysP)


NWORK = 32
TPW = T // NWORK
NSEL = PH * TK


GRP = 8


def _k5(values, vi, w):
    mesh = plsc.VectorSubcoreMesh(core_axis_name="c", subcore_axis_name="s")

    @functools.partial(
        pl.kernel,
        mesh=mesh,
        out_type=jax.ShapeDtypeStruct((T, E), jnp.float32),
        scratch_types=[
            pltpu.VMEM((TPW, NSEL), jnp.int32),
            pltpu.VMEM((TPW, NSEL * 16), jnp.float32),
            pltpu.VMEM((NSEL, E), jnp.float32),
            pltpu.VMEM((NSEL, E), jnp.float32),
            pltpu.VMEM((GRP, E), jnp.float32),
            pltpu.VMEM((GRP, E), jnp.float32),
            pltpu.SemaphoreType.DMA,
            pltpu.SemaphoreType.DMA,
            pltpu.SemaphoreType.DMA,
            pltpu.SemaphoreType.DMA,
        ],
    )
    def k5(values_hbm, vi_hbm, w_hbm, out_hbm,
           idx_v, w_v, rows0, rows1, ob0, ob1,
           sem_g0, sem_g1, sem_o0, sem_o1):
        wid = lax.axis_index("s") * 2 + lax.axis_index("c")
        base = wid * TPW
        rows = (rows0, rows1)
        sem_g = (sem_g0, sem_g1)
        obs = (ob0, ob1)
        sem_o = (sem_o0, sem_o1)
        pltpu.sync_copy(vi_hbm.at[pl.ds(base, TPW)], idx_v)
        pltpu.sync_copy(w_hbm.at[pl.ds(base, TPW)], w_v)
        pltpu.make_async_copy(values_hbm.at[idx_v.at[0]], rows0, sem_g0).start()

        def token(t, rows_cur, sem_cur, rows_nxt, sem_nxt, ob, tt):
            @pl.when(t + 1 < TPW)
            def _():
                pltpu.make_async_copy(
                    values_hbm.at[idx_v.at[t + 1]], rows_nxt, sem_nxt
                ).start()

            pltpu.make_async_copy(
                values_hbm.at[idx_v.at[t]], rows_cur, sem_cur
            ).wait()

            ws = [w_v[t, pl.ds(j * 16, 16)] for j in range(NSEL)]

            def cbody(c, carry2):
                seg = pl.ds(c * 16, 16)
                acc = ws[0] * rows_cur[0, seg]
                for j in range(1, NSEL):
                    acc = acc + ws[j] * rows_cur[j, seg]
                ob[tt, seg] = acc
                return carry2

            lax.fori_loop(0, E // 16, cbody, 0)

        def gpair(gp, carry):
            for gh in range(2):
                g0 = gp * 2 * GRP + gh * GRP

                @pl.when(gp > 0)
                def _():
                    pltpu.make_async_copy(
                        obs[gh], out_hbm.at[pl.ds(base + g0, GRP)], sem_o[gh]
                    ).wait()

                for tt in range(GRP):
                    t = g0 + tt
                    par = tt % 2
                    token(t, rows[par], sem_g[par], rows[1 - par],
                          sem_g[1 - par], obs[gh], tt)
                pltpu.make_async_copy(
                    obs[gh], out_hbm.at[pl.ds(base + g0, GRP)], sem_o[gh]
                ).start()
            return carry

        ngp = TPW // (2 * GRP)
        lax.fori_loop(0, ngp, gpair, 0)
        last = (ngp - 1) * 2 * GRP
        pltpu.make_async_copy(
            obs[0], out_hbm.at[pl.ds(base + last, GRP)], sem_o[0]
        ).wait()
        pltpu.make_async_copy(
            obs[1], out_hbm.at[pl.ds(base + last + GRP, GRP)], sem_o[1]
        ).wait()

    return k5(values, vi, w)


def _k6_body(a_ref, b_ref, o_ref):
    o_ref[...] = a_ref[...] + b_ref[...]


def _k6(a, b):
    return pl.pallas_call(
        _k6_body,
        grid=(NTB,),
        compiler_params=pltpu.CompilerParams(
            dimension_semantics=("parallel",)
        ),
        in_specs=[
            pl.BlockSpec((BT, E), lambda i: (i, 0)),
            pl.BlockSpec((BT, E), lambda i: (i, 0)),
        ],
        out_specs=pl.BlockSpec((BT, E), lambda i: (i, 0)),
        out_shape=jax.ShapeDtypeStruct((T, E), jnp.float32),
    )(a, b)


def kernel(hidden_states, ln1_g, ln1_b, Wqkv, bqkv, Wo, bo,
           ln2_g, ln2_b, Wq, bn_g, bn_b, pkm_keys, pkm_values):
    x2d = hidden_states.reshape(T, E)
    qkv = _k1(x2d, ln1_g.reshape(1, E), ln1_b.reshape(1, E),
              Wqkv.astype(jnp.bfloat16), bqkv.reshape(1, 3 * E))
    attn, ctx2d = _k2(qkv)
    hidden, queries, stats = _k3(ctx2d, Wo.astype(jnp.bfloat16),
                                 bo.reshape(1, E), x2d,
                                 ln2_g.reshape(1, E), ln2_b.reshape(1, E),
                                 Wq.astype(jnp.bfloat16))
    keysP = pkm_keys.transpose(2, 0, 1, 3)
    w_t, vi_t = _k4(queries, stats, bn_g.reshape(1, DQ), bn_b.reshape(1, DQ),
                    keysP)
    w_exp = jnp.broadcast_to(
        w_t.T[:, :, None], (T, NSEL, 16)
    ).reshape(T, NSEL * 16)
    pkm = _k5(pkm_values, vi_t.T, w_exp)
    out = _k6(hidden, pkm)
    return out.reshape(1, T, E), attn.reshape(1, H, T, T)


# speedup vs baseline: 1.0079x; 1.0079x over previous
import functools

import jax
import jax.numpy as jnp
from jax import lax
from jax.experimental import pallas as pl
from jax.experimental.pallas import tpu as pltpu
from jax.experimental.pallas import tpu_sc as plsc

T = 2048
E = 1024
H = 16
DH = 64
PH = 4
NK = 128
TK = 8
DQ = 1024
BT = 512
NTB = T // BT


def _k1_body(x_ref, g_ref, b_ref, w_ref, bias_ref, out_ref):
    x = x_ref[...]
    mu = jnp.mean(x, axis=1, keepdims=True)
    var = jnp.mean((x - mu) ** 2, axis=1, keepdims=True)
    xn = (x - mu) * lax.rsqrt(var + 1e-5) * g_ref[...] + b_ref[...]
    out_ref[...] = (
        jnp.dot(xn.astype(jnp.bfloat16), w_ref[...],
                preferred_element_type=jnp.float32)
        + bias_ref[...]
    ).astype(jnp.bfloat16)


def _k1(x, g, b, W, bias):
    return pl.pallas_call(
        _k1_body,
        grid=(NTB,),
        compiler_params=pltpu.CompilerParams(
            dimension_semantics=("parallel",)
        ),
        in_specs=[
            pl.BlockSpec((BT, E), lambda i: (i, 0)),
            pl.BlockSpec((1, E), lambda i: (0, 0)),
            pl.BlockSpec((1, E), lambda i: (0, 0)),
            pl.BlockSpec((E, 3 * E), lambda i: (0, 0)),
            pl.BlockSpec((1, 3 * E), lambda i: (0, 0)),
        ],
        out_specs=pl.BlockSpec((BT, 3 * E), lambda i: (i, 0)),
        out_shape=jax.ShapeDtypeStruct((T, 3 * E), jnp.bfloat16),
    )(x, g, b, W, bias)


BT2 = 512
NTB2 = T // BT2


def _k2_body(q_ref, k_ref, v_ref, attn_ref, ctx_ref):
    q2 = q_ref[...]
    k2 = k_ref[...]
    v2 = v_ref[...]
    ctx_parts = []
    for half in range(2):
        c = slice(half * DH, (half + 1) * DH)
        s = lax.dot_general(
            q2[:, c], k2[:, c], (((1,), (1,)), ((), ())),
            preferred_element_type=jnp.float32,
        ) * 0.125
        m = jnp.max(s, axis=1, keepdims=True)
        p = jnp.exp(s - m)
        inv = 1.0 / jnp.sum(p, axis=1, keepdims=True)
        attn_ref[half] = p * inv
        ctx = lax.dot_general(
            p.astype(jnp.bfloat16), v2[:, c], (((1,), (0,)), ((), ())),
            preferred_element_type=jnp.float32,
        )
        ctx_parts.append(ctx * inv)
    ctx_ref[...] = jnp.concatenate(ctx_parts, axis=1).astype(jnp.bfloat16)


def _k2(qkv):
    return pl.pallas_call(
        _k2_body,
        grid=(H // 2, NTB2),
        compiler_params=pltpu.CompilerParams(
            dimension_semantics=("parallel", "parallel")
        ),
        in_specs=[
            pl.BlockSpec((BT2, 2 * DH), lambda h, i: (i, h)),
            pl.BlockSpec((T, 2 * DH), lambda h, i: (0, H // 2 + h)),
            pl.BlockSpec((T, 2 * DH), lambda h, i: (0, H + h)),
        ],
        out_specs=[
            pl.BlockSpec((2, BT2, T), lambda h, i: (h, i, 0)),
            pl.BlockSpec((BT2, 2 * DH), lambda h, i: (i, h)),
        ],
        out_shape=[
            jax.ShapeDtypeStruct((H, T, T), jnp.float32),
            jax.ShapeDtypeStruct((T, E), jnp.bfloat16),
        ],
    )(qkv, qkv, qkv)


def _k3_body(ctx_ref, wo_ref, bo_ref, res_ref, g_ref, b_ref, wq_ref,
             hid_ref, q_ref, st_ref):
    i = pl.program_id(0)
    attn_out = (
        jnp.dot(ctx_ref[...], wo_ref[...], preferred_element_type=jnp.float32)
        + bo_ref[...]
    )
    hid = attn_out + res_ref[...]
    hid_ref[...] = hid
    mu = jnp.mean(hid, axis=1, keepdims=True)
    var = jnp.mean((hid - mu) ** 2, axis=1, keepdims=True)
    xn = (hid - mu) * lax.rsqrt(var + 1e-5) * g_ref[...] + b_ref[...]
    q = jnp.dot(xn.astype(jnp.bfloat16), wq_ref[...],
                preferred_element_type=jnp.float32)
    q_ref[...] = q
    s1 = jnp.sum(q, axis=0, keepdims=True)
    s2 = jnp.sum(q * q, axis=0, keepdims=True)
    blk = jnp.concatenate([s1, s2], axis=0)

    @pl.when(i == 0)
    def _():
        st_ref[...] = blk

    @pl.when(i != 0)
    def _():
        st_ref[...] = st_ref[...] + blk


def _k3(ctx, Wo, bo, resid, g, b, Wq):
    return pl.pallas_call(
        _k3_body,
        grid=(NTB,),
        in_specs=[
            pl.BlockSpec((BT, E), lambda i: (i, 0)),
            pl.BlockSpec((E, E), lambda i: (0, 0)),
            pl.BlockSpec((1, E), lambda i: (0, 0)),
            pl.BlockSpec((BT, E), lambda i: (i, 0)),
            pl.BlockSpec((1, E), lambda i: (0, 0)),
            pl.BlockSpec((1, E), lambda i: (0, 0)),
            pl.BlockSpec((E, DQ), lambda i: (0, 0)),
        ],
        out_specs=[
            pl.BlockSpec((BT, E), lambda i: (i, 0)),
            pl.BlockSpec((BT, DQ), lambda i: (i, 0)),
            pl.BlockSpec((2, DQ), lambda i: (0, 0)),
        ],
        out_shape=[
            jax.ShapeDtypeStruct((T, E), jnp.float32),
            jax.ShapeDtypeStruct((T, DQ), jnp.float32),
            jax.ShapeDtypeStruct((2, DQ), jnp.float32),
        ],
    )(ctx, Wo, bo, resid, g, b, Wq)


def _topk8t(s, iota):
    n = s.shape[0]
    cur = s
    vals, poss = [], []
    for _ in range(TK):
        m = jnp.max(cur, axis=0, keepdims=True)
        p = jnp.min(
            jnp.where(cur == m, iota, jnp.int32(n)), axis=0, keepdims=True
        )
        vals.append(m)
        poss.append(p)
        cur = jnp.where(iota == p, jnp.float32(-3e38), cur)
    return jnp.concatenate(vals, axis=0), jnp.concatenate(poss, axis=0)


def _k4_body(q_ref, st_ref, g_ref, b_ref, keys_ref, w_ref, vi_ref):
    q = q_ref[...]
    st = st_ref[...]
    mu = st[0:1, :] * (1.0 / T)
    ms = st[1:2, :] * (1.0 / T)
    var = ms - mu * mu
    qn = (q - mu) * lax.rsqrt(var + 1e-5) * g_ref[...] + b_ref[...]
    iota_nk = lax.broadcasted_iota(jnp.int32, (NK, BT), 0)
    iota_cc = lax.broadcasted_iota(jnp.int32, (TK * TK, BT), 0)
    w_parts, vi_parts = [], []
    for h in range(PH):
        q1 = qn[:, h * NK:(h + 1) * NK]
        q2 = qn[:, PH * NK + h * NK: PH * NK + (h + 1) * NK]
        d1 = lax.dot_general(
            keys_ref[0, h], q1, (((1,), (1,)), ((), ())),
            preferred_element_type=jnp.float32,
        )
        d2 = lax.dot_general(
            keys_ref[1, h], q2, (((1,), (1,)), ((), ())),
            preferred_element_type=jnp.float32,
        )
        s1v, i1v = _topk8t(d1, iota_nk)
        s2v, i2v = _topk8t(d2, iota_nk)
        combs, combi = [], []
        for a in range(TK):
            combs.append(s1v[a:a + 1, :] + s2v)
            combi.append(i1v[a:a + 1, :] * NK + i2v)
        comb = jnp.concatenate(combs, axis=0)
        ci = jnp.concatenate(combi, axis=0)
        fs, fpos = _topk8t(comb, iota_cc)
        sels = []
        for j in range(TK):
            pj = fpos[j:j + 1, :]
            sels.append(
                jnp.min(
                    jnp.where(iota_cc == pj, ci, jnp.int32(1 << 30)),
                    axis=0,
                    keepdims=True,
                )
            )
        vi_h = jnp.concatenate(sels, axis=0)
        ex = jnp.exp(fs - fs[0:1, :])
        wgt = ex * (1.0 / jnp.sum(ex, axis=0, keepdims=True))
        w_parts.append(wgt)
        vi_parts.append(vi_h)
    w_ref[...] = jnp.concatenate(w_parts, axis=0)
    vi_ref[...] = jnp.concatenate(vi_parts, axis=0)


def _k4(queries, stats, g, b, keysP):
    return pl.pallas_call(
        _k4_body,
        grid=(NTB,),
        compiler_params=pltpu.CompilerParams(
            dimension_semantics=("parallel",)
        ),
        in_specs=[
            pl.BlockSpec((BT, DQ), lambda i: (i, 0)),
            pl.BlockSpec((2, DQ), lambda i: (0, 0)),
            pl.BlockSpec((1, DQ), lambda i: (0, 0)),
            pl.BlockSpec((1, DQ), lambda i: (0, 0)),
            pl.BlockSpec((2, PH, NK, NK), lambda i: (0, 0, 0, 0)),
        ],
        out_specs=[
            pl.BlockSpec((PH * TK, BT), lambda i: (0, i)),
            pl.BlockSpec((PH * TK, BT), lambda i: (0, i)),
        ],
        out_shape=[
            jax.ShapeDtypeStruct((PH * TK, T), jnp.float32),
            jax.ShapeDtypeStruct((PH * TK, T), jnp.int32),
        ],
    )(queries, stats, g, b, keysP)


NWORK = 32
TPW = T // NWORK
NSEL = PH * TK


GRP = 8


def _k5(values, vi, w):
    mesh = plsc.VectorSubcoreMesh(core_axis_name="c", subcore_axis_name="s")

    @functools.partial(
        pl.kernel,
        mesh=mesh,
        out_type=jax.ShapeDtypeStruct((T, E), jnp.float32),
        scratch_types=[
            pltpu.VMEM((TPW, NSEL), jnp.int32),
            pltpu.VMEM((TPW, NSEL * 16), jnp.float32),
            pltpu.VMEM((NSEL, E), jnp.float32),
            pltpu.VMEM((NSEL, E), jnp.float32),
            pltpu.VMEM((GRP, E), jnp.float32),
            pltpu.VMEM((GRP, E), jnp.float32),
            pltpu.SemaphoreType.DMA,
            pltpu.SemaphoreType.DMA,
            pltpu.SemaphoreType.DMA,
            pltpu.SemaphoreType.DMA,
        ],
    )
    def k5(values_hbm, vi_hbm, w_hbm, out_hbm,
           idx_v, w_v, rows0, rows1, ob0, ob1,
           sem_g0, sem_g1, sem_o0, sem_o1):
        wid = lax.axis_index("s") * 2 + lax.axis_index("c")
        base = wid * TPW
        rows = (rows0, rows1)
        sem_g = (sem_g0, sem_g1)
        obs = (ob0, ob1)
        sem_o = (sem_o0, sem_o1)
        pltpu.sync_copy(vi_hbm.at[pl.ds(base, TPW)], idx_v)
        pltpu.sync_copy(w_hbm.at[pl.ds(base, TPW)], w_v)
        pltpu.make_async_copy(values_hbm.at[idx_v.at[0]], rows0, sem_g0).start()

        def token(t, rows_cur, sem_cur, rows_nxt, sem_nxt, ob, tt):
            @pl.when(t + 1 < TPW)
            def _():
                pltpu.make_async_copy(
                    values_hbm.at[idx_v.at[t + 1]], rows_nxt, sem_nxt
                ).start()

            pltpu.make_async_copy(
                values_hbm.at[idx_v.at[t]], rows_cur, sem_cur
            ).wait()

            ws = [w_v[t, pl.ds(j * 16, 16)] for j in range(NSEL)]

            def cbody(c, carry2):
                seg = pl.ds(c * 16, 16)
                acc = ws[0] * rows_cur[0, seg]
                for j in range(1, NSEL):
                    acc = acc + ws[j] * rows_cur[j, seg]
                ob[tt, seg] = acc
                return carry2

            lax.fori_loop(0, E // 16, cbody, 0)

        def gpair(gp, carry):
            for gh in range(2):
                g0 = gp * 2 * GRP + gh * GRP

                @pl.when(gp > 0)
                def _():
                    pltpu.make_async_copy(
                        obs[gh], out_hbm.at[pl.ds(base + g0, GRP)], sem_o[gh]
                    ).wait()

                for tt in range(GRP):
                    t = g0 + tt
                    par = tt % 2
                    token(t, rows[par], sem_g[par], rows[1 - par],
                          sem_g[1 - par], obs[gh], tt)
                pltpu.make_async_copy(
                    obs[gh], out_hbm.at[pl.ds(base + g0, GRP)], sem_o[gh]
                ).start()
            return carry

        ngp = TPW // (2 * GRP)
        lax.fori_loop(0, ngp, gpair, 0)
        last = (ngp - 1) * 2 * GRP
        pltpu.make_async_copy(
            obs[0], out_hbm.at[pl.ds(base + last, GRP)], sem_o[0]
        ).wait()
        pltpu.make_async_copy(
            obs[1], out_hbm.at[pl.ds(base + last + GRP, GRP)], sem_o[1]
        ).wait()

    return k5(values, vi, w)


def _k6_body(a_ref, b_ref, o_ref):
    o_ref[...] = a_ref[...] + b_ref[...]


def _k6(a, b):
    return pl.pallas_call(
        _k6_body,
        grid=(NTB,),
        compiler_params=pltpu.CompilerParams(
            dimension_semantics=("parallel",)
        ),
        in_specs=[
            pl.BlockSpec((BT, E), lambda i: (i, 0)),
            pl.BlockSpec((BT, E), lambda i: (i, 0)),
        ],
        out_specs=pl.BlockSpec((BT, E), lambda i: (i, 0)),
        out_shape=jax.ShapeDtypeStruct((T, E), jnp.float32),
    )(a, b)


def kernel(hidden_states, ln1_g, ln1_b, Wqkv, bqkv, Wo, bo,
           ln2_g, ln2_b, Wq, bn_g, bn_b, pkm_keys, pkm_values):
    x2d = hidden_states.reshape(T, E)
    qkv = _k1(x2d, ln1_g.reshape(1, E), ln1_b.reshape(1, E),
              Wqkv.astype(jnp.bfloat16), bqkv.reshape(1, 3 * E))
    attn, ctx2d = _k2(qkv)
    hidden, queries, stats = _k3(ctx2d, Wo.astype(jnp.bfloat16),
                                 bo.reshape(1, E), x2d,
                                 ln2_g.reshape(1, E), ln2_b.reshape(1, E),
                                 Wq.astype(jnp.bfloat16))
    keysP = pkm_keys.transpose(2, 0, 1, 3)
    w_t, vi_t = _k4(queries, stats, bn_g.reshape(1, DQ), bn_b.reshape(1, DQ),
                    keysP)
    w_exp = jnp.broadcast_to(
        w_t.T[:, :, None], (T, NSEL, 16)
    ).reshape(T, NSEL * 16)
    pkm = _k5(pkm_values, vi_t.T, w_exp)
    out = _k6(hidden, pkm)
    return out.reshape(1, T, E), attn.reshape(1, H, T, T)
